# Initial kernel scaffold; baseline (speedup 1.0000x reference)
import jax
import jax.numpy as jnp
from jax.experimental import pallas as pl


def kernel(y_pred, y_true):
    raise NotImplementedError("write your pallas kernel here")


# speedup vs baseline: 1.2354x; 1.2354x over previous
import jax
import jax.numpy as jnp
from jax import lax
from jax.experimental import pallas as pl
from jax.experimental.pallas import tpu as pltpu

_C = 1000
_CPAD = 1024
_EPS = 1e-07
_B = 16384
_TB = 512


def _f1_kernel(yp_ref, yt_ref, out_ref, ht_ref, hp_ref, tp_ref):
    i = pl.program_id(0)
    nsteps = pl.num_programs(0)

    @pl.when(i == 0)
    def _init():
        ht_ref[...] = jnp.zeros_like(ht_ref)
        hp_ref[...] = jnp.zeros_like(hp_ref)
        tp_ref[...] = jnp.zeros_like(tp_ref)

    x = yp_ref[...]
    t = yt_ref[0, 0, :]
    m = jnp.max(x, axis=1, keepdims=True)
    cls = lax.broadcasted_iota(jnp.int32, x.shape, 1)
    pred = jnp.min(jnp.where(x == m, cls, _C), axis=1)

    cpad = lax.broadcasted_iota(jnp.int32, (_TB, _CPAD), 1)
    oh_p = (pred[:, None] == cpad).astype(jnp.float32)
    oh_t = (t[:, None] == cpad).astype(jnp.float32)
    correct = (pred == t).astype(jnp.float32)
    hp_ref[...] += jnp.sum(oh_p, axis=0, keepdims=True)
    ht_ref[...] += jnp.sum(oh_t, axis=0, keepdims=True)
    tp_ref[...] += jnp.sum(oh_p * correct[:, None], axis=0, keepdims=True)

    @pl.when(i == nsteps - 1)
    def _final():
        tp = tp_ref[0, :]
        ht = ht_ref[0, :]
        hp = hp_ref[0, :]
        sens = jnp.sum(tp / (hp + _EPS)) / _C
        prec = jnp.sum(tp / (ht + _EPS)) / _C
        f1 = 2.0 * (prec * sens) / (prec + sens + _EPS)
        out_ref[...] = jnp.broadcast_to(f1, (1, 1))


def kernel(y_pred, y_true):
    nb = _B // _TB
    yt3 = y_true.reshape(nb, 1, _TB)
    out = pl.pallas_call(
        _f1_kernel,
        grid=(nb,),
        in_specs=[
            pl.BlockSpec((_TB, _C), lambda i: (i, 0)),
            pl.BlockSpec((1, 1, _TB), lambda i: (i, 0, 0)),
        ],
        out_specs=pl.BlockSpec((1, 1), lambda i: (0, 0)),
        out_shape=jax.ShapeDtypeStruct((1, 1), jnp.float32),
        scratch_shapes=[
            pltpu.VMEM((1, _CPAD), jnp.float32),
            pltpu.VMEM((1, _CPAD), jnp.float32),
            pltpu.VMEM((1, _CPAD), jnp.float32),
        ],
    )(y_pred, yt3)
    return out[0, 0]
